# speedup vs baseline: 75.9810x; 75.9810x over previous
import functools

import jax
import jax.numpy as jnp
from jax import lax
from jax.experimental import pallas as pl

_IOU_T = 0.7
_B = 512


def _iou_tile(rx1, ry1, rx2, ry2, ra, cx1, cy1, cx2, cy2, ca):
    ltx = jnp.maximum(rx1, cx1)
    lty = jnp.maximum(ry1, cy1)
    rbx = jnp.minimum(rx2, cx2)
    rby = jnp.minimum(ry2, cy2)
    wx = jnp.maximum(rbx - ltx, 0.0)
    wy = jnp.maximum(rby - lty, 0.0)
    inter = wx * wy
    union = ra + ca - inter
    return inter / jnp.maximum(union, 1e-8)


def _nms_body(x1c, y1c, x2c, y2c, ac, x1r, y1r, x2r, y2r, ar, keep_ref):
    NP = x1r.shape[1]
    NB = NP // _B
    keep_ref[...] = jnp.ones((1, NP), jnp.float32)

    ii = lax.broadcasted_iota(jnp.int32, (_B, _B), 0)
    jj = lax.broadcasted_iota(jnp.int32, (_B, _B), 1)
    tri = ii < jj

    def outer(i, carry):
        base = i * _B
        rx1 = x1c[pl.ds(base, _B), :]
        ry1 = y1c[pl.ds(base, _B), :]
        rx2 = x2c[pl.ds(base, _B), :]
        ry2 = y2c[pl.ds(base, _B), :]
        ra = ac[pl.ds(base, _B), :]
        bx1 = x1r[:, pl.ds(base, _B)]
        by1 = y1r[:, pl.ds(base, _B)]
        bx2 = x2r[:, pl.ds(base, _B)]
        by2 = y2r[:, pl.ds(base, _B)]
        ba = ar[:, pl.ds(base, _B)]

        iou_ii = _iou_tile(rx1, ry1, rx2, ry2, ra, bx1, by1, bx2, by2, ba)
        sup_ii = jnp.where((iou_ii > _IOU_T) & tri, 1.0, 0.0)

        a0 = keep_ref[:, pl.ds(base, _B)]

        def fix_cond(st):
            return st[1]

        def fix_body(st):
            kept, _ = st
            supp = jnp.dot(kept, sup_ii, preferred_element_type=jnp.float32)
            new = jnp.where(supp > 0.5, 0.0, a0)
            return (new, jnp.any(new != kept))

        kept, _ = lax.while_loop(fix_cond, fix_body, (a0, jnp.bool_(True)))
        keep_ref[:, pl.ds(base, _B)] = kept

        def cross(c, kept_):
            cb = c * _B
            cx1 = x1r[:, pl.ds(cb, _B)]
            cy1 = y1r[:, pl.ds(cb, _B)]
            cx2 = x2r[:, pl.ds(cb, _B)]
            cy2 = y2r[:, pl.ds(cb, _B)]
            ca = ar[:, pl.ds(cb, _B)]
            iou_ic = _iou_tile(rx1, ry1, rx2, ry2, ra, cx1, cy1, cx2, cy2, ca)
            sup = jnp.where(iou_ic > _IOU_T, 1.0, 0.0)
            supp = jnp.dot(kept_, sup, preferred_element_type=jnp.float32)
            cur = keep_ref[:, pl.ds(cb, _B)]
            keep_ref[:, pl.ds(cb, _B)] = jnp.where(supp > 0.5, 0.0, cur)
            return kept_

        lax.fori_loop(i + 1, NB, cross, kept)
        return carry

    lax.fori_loop(0, NB, outer, 0)


@jax.jit
def kernel(boxes, scores):
    n = boxes.shape[0]
    order = jnp.argsort(-scores)
    boxes_s = boxes[order]

    NP = ((n + _B - 1) // _B) * _B
    pad = NP - n
    boxes_p = jnp.concatenate(
        [boxes_s, jnp.zeros((pad, 4), jnp.float32)], axis=0)

    x1 = boxes_p[:, 0]
    y1 = boxes_p[:, 1]
    x2 = boxes_p[:, 2]
    y2 = boxes_p[:, 3]
    area = (x2 - x1) * (y2 - y1)

    col = lambda v: v.reshape(NP, 1)
    row = lambda v: v.reshape(1, NP)

    keep_f = pl.pallas_call(
        _nms_body,
        out_shape=jax.ShapeDtypeStruct((1, NP), jnp.float32),
    )(col(x1), col(y1), col(x2), col(y2), col(area),
      row(x1), row(y1), row(x2), row(y2), row(area))

    keep_sorted = keep_f[0, :n] > 0.5
    keep = jnp.zeros((n,), jnp.bool_).at[order].set(keep_sorted)
    kept_scores = jnp.where(keep, scores, 0.0)
    return kept_scores, keep


# speedup vs baseline: 79.8179x; 1.0505x over previous
import functools

import jax
import jax.numpy as jnp
from jax import lax
from jax.experimental import pallas as pl

_IOU_T = 0.7
_B = 512
_CB = 1024


def _iou_tile(rx1, ry1, rx2, ry2, ra, cx1, cy1, cx2, cy2, ca):
    ltx = jnp.maximum(rx1, cx1)
    lty = jnp.maximum(ry1, cy1)
    rbx = jnp.minimum(rx2, cx2)
    rby = jnp.minimum(ry2, cy2)
    wx = jnp.maximum(rbx - ltx, 0.0)
    wy = jnp.maximum(rby - lty, 0.0)
    inter = wx * wy
    union = ra + ca - inter
    return inter / jnp.maximum(union, 1e-8)


def _nms_body(x1c, y1c, x2c, y2c, ac, x1r, y1r, x2r, y2r, ar, keep_ref):
    NP = x1r.shape[1]
    NB = NP // _B
    keep_ref[...] = jnp.ones((1, NP), jnp.float32)

    ii = lax.broadcasted_iota(jnp.int32, (_B, _B), 0)
    jj = lax.broadcasted_iota(jnp.int32, (_B, _B), 1)
    tri = ii < jj

    def outer(i, carry):
        base = i * _B
        rx1 = x1c[pl.ds(base, _B), :]
        ry1 = y1c[pl.ds(base, _B), :]
        rx2 = x2c[pl.ds(base, _B), :]
        ry2 = y2c[pl.ds(base, _B), :]
        ra = ac[pl.ds(base, _B), :]
        bx1 = x1r[:, pl.ds(base, _B)]
        by1 = y1r[:, pl.ds(base, _B)]
        bx2 = x2r[:, pl.ds(base, _B)]
        by2 = y2r[:, pl.ds(base, _B)]
        ba = ar[:, pl.ds(base, _B)]

        iou_ii = _iou_tile(rx1, ry1, rx2, ry2, ra, bx1, by1, bx2, by2, ba)
        sup_ii = jnp.where((iou_ii > _IOU_T) & tri, 1.0, 0.0)

        a0 = keep_ref[:, pl.ds(base, _B)]

        def fix_cond(st):
            return st[1]

        def fix_body(st):
            kept, _ = st
            supp = jnp.dot(kept, sup_ii, preferred_element_type=jnp.float32)
            new = jnp.where(supp > 0.5, 0.0, a0)
            return (new, jnp.any(new != kept))

        kept, _ = lax.while_loop(fix_cond, fix_body, (a0, jnp.bool_(True)))
        keep_ref[:, pl.ds(base, _B)] = kept

        def cross(c, kept_):
            cb = c * _CB
            cx1 = x1r[:, pl.ds(cb, _CB)]
            cy1 = y1r[:, pl.ds(cb, _CB)]
            cx2 = x2r[:, pl.ds(cb, _CB)]
            cy2 = y2r[:, pl.ds(cb, _CB)]
            ca = ar[:, pl.ds(cb, _CB)]
            iou_ic = _iou_tile(rx1, ry1, rx2, ry2, ra, cx1, cy1, cx2, cy2, ca)
            sup = jnp.where(iou_ic > _IOU_T, 1.0, 0.0)
            supp = jnp.dot(kept_, sup, preferred_element_type=jnp.float32)
            col = cb + lax.broadcasted_iota(jnp.int32, (1, _CB), 1)
            valid = col >= base + _B
            cur = keep_ref[:, pl.ds(cb, _CB)]
            keep_ref[:, pl.ds(cb, _CB)] = jnp.where(
                (supp > 0.5) & valid, 0.0, cur)
            return kept_

        lax.fori_loop((base + _B) // _CB, NP // _CB, cross, kept)
        return carry

    lax.fori_loop(0, NB, outer, 0)


@jax.jit
def kernel(boxes, scores):
    n = boxes.shape[0]
    order = jnp.argsort(-scores)
    boxes_s = boxes[order]

    NP = ((n + _CB - 1) // _CB) * _CB
    pad = NP - n
    boxes_p = jnp.concatenate(
        [boxes_s, jnp.zeros((pad, 4), jnp.float32)], axis=0)

    x1 = boxes_p[:, 0]
    y1 = boxes_p[:, 1]
    x2 = boxes_p[:, 2]
    y2 = boxes_p[:, 3]
    area = (x2 - x1) * (y2 - y1)

    col = lambda v: v.reshape(NP, 1)
    row = lambda v: v.reshape(1, NP)

    keep_f = pl.pallas_call(
        _nms_body,
        out_shape=jax.ShapeDtypeStruct((1, NP), jnp.float32),
    )(col(x1), col(y1), col(x2), col(y2), col(area),
      row(x1), row(y1), row(x2), row(y2), row(area))

    keep_sorted = keep_f[0, :n] > 0.5
    keep = jnp.zeros((n,), jnp.bool_).at[order].set(keep_sorted)
    kept_scores = jnp.where(keep, scores, 0.0)
    return kept_scores, keep


# speedup vs baseline: 88.2880x; 1.1061x over previous
import jax
import jax.numpy as jnp
from jax import lax
from jax.experimental import pallas as pl
from jax.experimental.pallas import tpu as pltpu

_IOU_T = 0.7
_B = 512
_CB = 1024


def _iou_tile(rx1, ry1, rx2, ry2, ra, cx1, cy1, cx2, cy2, ca):
    ltx = jnp.maximum(rx1, cx1)
    lty = jnp.maximum(ry1, cy1)
    rbx = jnp.minimum(rx2, cx2)
    rby = jnp.minimum(ry2, cy2)
    wx = jnp.maximum(rbx - ltx, 0.0)
    wy = jnp.maximum(rby - lty, 0.0)
    inter = wx * wy
    union = ra + ca - inter
    return inter / jnp.maximum(union, 1e-8)


def _nms_body(p_ref, keep_ref, area_ref):
    NP = p_ref.shape[1]
    NB = NP // _B
    keep_ref[...] = jnp.ones((1, NP), jnp.float32)
    area_ref[...] = ((p_ref[2:3, :] - p_ref[0:1, :]) *
                     (p_ref[3:4, :] - p_ref[1:2, :]))

    ii = lax.broadcasted_iota(jnp.int32, (_B, _B), 0)
    jj = lax.broadcasted_iota(jnp.int32, (_B, _B), 1)
    tri = ii < jj

    def outer(i, carry):
        base = i * _B
        bx1 = p_ref[0:1, pl.ds(base, _B)]
        by1 = p_ref[1:2, pl.ds(base, _B)]
        bx2 = p_ref[2:3, pl.ds(base, _B)]
        by2 = p_ref[3:4, pl.ds(base, _B)]
        ba = area_ref[:, pl.ds(base, _B)]
        rx1 = jnp.transpose(bx1)
        ry1 = jnp.transpose(by1)
        rx2 = jnp.transpose(bx2)
        ry2 = jnp.transpose(by2)
        ra = (rx2 - rx1) * (ry2 - ry1)

        iou_ii = _iou_tile(rx1, ry1, rx2, ry2, ra, bx1, by1, bx2, by2, ba)
        sup_ii = jnp.where((iou_ii > _IOU_T) & tri, 1.0, 0.0)

        a0 = keep_ref[:, pl.ds(base, _B)]

        def fix_cond(st):
            return st[1]

        def fix_body(st):
            kept, _ = st
            supp = jnp.dot(kept, sup_ii, preferred_element_type=jnp.float32)
            new = jnp.where(supp > 0.5, 0.0, a0)
            return (new, jnp.any(new != kept))

        kept, _ = lax.while_loop(fix_cond, fix_body, (a0, jnp.bool_(True)))
        keep_ref[:, pl.ds(base, _B)] = kept

        def cross(c, kept_):
            cb = c * _CB
            cx1 = p_ref[0:1, pl.ds(cb, _CB)]
            cy1 = p_ref[1:2, pl.ds(cb, _CB)]
            cx2 = p_ref[2:3, pl.ds(cb, _CB)]
            cy2 = p_ref[3:4, pl.ds(cb, _CB)]
            ca = area_ref[:, pl.ds(cb, _CB)]
            iou_ic = _iou_tile(rx1, ry1, rx2, ry2, ra, cx1, cy1, cx2, cy2, ca)
            sup = jnp.where(iou_ic > _IOU_T, 1.0, 0.0)
            supp = jnp.dot(kept_, sup, preferred_element_type=jnp.float32)
            col = cb + lax.broadcasted_iota(jnp.int32, (1, _CB), 1)
            valid = col >= base + _B
            cur = keep_ref[:, pl.ds(cb, _CB)]
            keep_ref[:, pl.ds(cb, _CB)] = jnp.where(
                (supp > 0.5) & valid, 0.0, cur)
            return kept_

        lax.fori_loop((base + _B) // _CB, NP // _CB, cross, kept)
        return carry

    lax.fori_loop(0, NB, outer, 0)


@jax.jit
def kernel(boxes, scores):
    n = boxes.shape[0]
    order = jnp.argsort(-scores)
    boxes_s = boxes[order]

    NP = ((n + _CB - 1) // _CB) * _CB
    p8 = jnp.zeros((8, NP), jnp.float32).at[:4, :n].set(boxes_s.T)

    keep_f = pl.pallas_call(
        _nms_body,
        out_shape=jax.ShapeDtypeStruct((1, NP), jnp.float32),
        scratch_shapes=[pltpu.VMEM((1, NP), jnp.float32)],
    )(p8)

    keep_sorted = keep_f[0, :n] > 0.5
    keep = jnp.zeros((n,), jnp.bool_).at[order].set(keep_sorted)
    kept_scores = jnp.where(keep, scores, 0.0)
    return kept_scores, keep


# speedup vs baseline: 89.8827x; 1.0181x over previous
import jax
import jax.numpy as jnp
from jax import lax
from jax.experimental import pallas as pl
from jax.experimental.pallas import tpu as pltpu

_IOU_T = 0.7
_B = 512
_CB = 2048


def _iou_tile(rx1, ry1, rx2, ry2, ra, cx1, cy1, cx2, cy2, ca):
    ltx = jnp.maximum(rx1, cx1)
    lty = jnp.maximum(ry1, cy1)
    rbx = jnp.minimum(rx2, cx2)
    rby = jnp.minimum(ry2, cy2)
    wx = jnp.maximum(rbx - ltx, 0.0)
    wy = jnp.maximum(rby - lty, 0.0)
    inter = wx * wy
    union = ra + ca - inter
    return inter / jnp.maximum(union, 1e-8)


def _nms_body(p_ref, keep_ref, area_ref):
    NP = p_ref.shape[1]
    NB = NP // _B
    keep_ref[...] = jnp.ones((1, NP), jnp.float32)
    area_ref[...] = ((p_ref[2:3, :] - p_ref[0:1, :]) *
                     (p_ref[3:4, :] - p_ref[1:2, :]))

    ii = lax.broadcasted_iota(jnp.int32, (_B, _B), 0)
    jj = lax.broadcasted_iota(jnp.int32, (_B, _B), 1)
    tri = ii < jj

    def outer(i, carry):
        base = i * _B
        bx1 = p_ref[0:1, pl.ds(base, _B)]
        by1 = p_ref[1:2, pl.ds(base, _B)]
        bx2 = p_ref[2:3, pl.ds(base, _B)]
        by2 = p_ref[3:4, pl.ds(base, _B)]
        ba = area_ref[:, pl.ds(base, _B)]
        rx1 = jnp.transpose(bx1)
        ry1 = jnp.transpose(by1)
        rx2 = jnp.transpose(bx2)
        ry2 = jnp.transpose(by2)
        ra = (rx2 - rx1) * (ry2 - ry1)

        iou_ii = _iou_tile(rx1, ry1, rx2, ry2, ra, bx1, by1, bx2, by2, ba)
        sup_ii = jnp.where((iou_ii > _IOU_T) & tri, 1.0, 0.0)

        a0 = keep_ref[:, pl.ds(base, _B)]

        def fix_cond(st):
            return st[1]

        def fix_body(st):
            kept, _ = st
            supp = jnp.dot(kept, sup_ii, preferred_element_type=jnp.float32)
            new = jnp.where(supp > 0.5, 0.0, a0)
            return (new, jnp.any(new != kept))

        kept, _ = lax.while_loop(fix_cond, fix_body, (a0, jnp.bool_(True)))
        keep_ref[:, pl.ds(base, _B)] = kept

        def cross(c, kept_):
            cb = c * _CB
            cx1 = p_ref[0:1, pl.ds(cb, _CB)]
            cy1 = p_ref[1:2, pl.ds(cb, _CB)]
            cx2 = p_ref[2:3, pl.ds(cb, _CB)]
            cy2 = p_ref[3:4, pl.ds(cb, _CB)]
            ca = area_ref[:, pl.ds(cb, _CB)]
            iou_ic = _iou_tile(rx1, ry1, rx2, ry2, ra, cx1, cy1, cx2, cy2, ca)
            sup = jnp.where(iou_ic > _IOU_T, 1.0, 0.0)
            supp = jnp.dot(kept_, sup, preferred_element_type=jnp.float32)
            col = cb + lax.broadcasted_iota(jnp.int32, (1, _CB), 1)
            valid = col >= base + _B
            cur = keep_ref[:, pl.ds(cb, _CB)]
            keep_ref[:, pl.ds(cb, _CB)] = jnp.where(
                (supp > 0.5) & valid, 0.0, cur)
            return kept_

        lax.fori_loop((base + _B) // _CB, NP // _CB, cross, kept)
        return carry

    lax.fori_loop(0, NB, outer, 0)


@jax.jit
def kernel(boxes, scores):
    n = boxes.shape[0]
    order = jnp.argsort(-scores)
    boxes_s = boxes[order]

    NP = ((n + _CB - 1) // _CB) * _CB
    p8 = jnp.zeros((8, NP), jnp.float32).at[:4, :n].set(boxes_s.T)

    keep_f = pl.pallas_call(
        _nms_body,
        out_shape=jax.ShapeDtypeStruct((1, NP), jnp.float32),
        scratch_shapes=[pltpu.VMEM((1, NP), jnp.float32)],
    )(p8)

    keep_sorted = keep_f[0, :n] > 0.5
    keep = jnp.zeros((n,), jnp.bool_).at[order].set(keep_sorted)
    kept_scores = jnp.where(keep, scores, 0.0)
    return kept_scores, keep
